# baseline (speedup 1.0000x reference)
import functools

import jax
import jax.numpy as jnp
from jax import lax
from jax.experimental import pallas as pl
from jax.experimental.pallas import tpu as pltpu
from jax.experimental.pallas import tpu_sc as plsc

BATCH = 4
SEQ = 2048
D = 128
NC, NS = 2, 16
NW = NC * NS
CHUNK = SEQ // NW
N_CHUNKS = BATCH


def _emb_body(idx_hbm, tok_hbm, pos_hbm, out_hbm, idx_v, rows_v, pos_v,
              psem, wsem, *gsems):
    c = lax.axis_index("c")
    s = lax.axis_index("s")
    wid = s * NC + c
    t0 = wid * CHUNK

    pltpu.sync_copy(idx_hbm.at[0, pl.ds(t0, CHUNK)], idx_v.at[0])
    gcps = [pltpu.async_copy(tok_hbm.at[idx_v.at[0]],
                             rows_v.at[pl.ds(0, CHUNK)], gsems[0])]
    for k in range(1, N_CHUNKS):
        pltpu.sync_copy(idx_hbm.at[k, pl.ds(t0, CHUNK)], idx_v.at[k])
        gcps.append(
            pltpu.async_copy(tok_hbm.at[idx_v.at[k]],
                             rows_v.at[pl.ds(k * CHUNK, CHUNK)], gsems[k]))

    pos_cp = pltpu.async_copy(pos_hbm.at[0, pl.ds(t0, CHUNK)], pos_v, psem)
    pos_cp.wait()

    HALF = CHUNK // 2
    wcps = []
    for k in range(N_CHUNKS):
        gcps[k].wait()

        for h in range(2):
            def add_row(i, carry):
                for j in range(D // 16):
                    sl = pl.ds(j * 16, 16)
                    plsc.addupdate(rows_v.at[k * CHUNK + h * HALF + i, sl],
                                   pos_v[h * HALF + i, sl])
                return carry

            lax.fori_loop(0, HALF, add_row, 0, unroll=2)
            wcps.append(
                pltpu.async_copy(
                    rows_v.at[pl.ds(k * CHUNK + h * HALF, HALF)],
                    out_hbm.at[k, pl.ds(t0 + h * HALF, HALF)], wsem))
    for cp in wcps:
        cp.wait()


@jax.jit
def _emb(idx, tok_emb, pos_emb):
    mesh = plsc.VectorSubcoreMesh(core_axis_name="c", subcore_axis_name="s")
    f = functools.partial(
        pl.kernel,
        mesh=mesh,
        out_type=jax.ShapeDtypeStruct((BATCH, SEQ, D), jnp.float32),
        scratch_types=[
            pltpu.VMEM((N_CHUNKS, CHUNK), jnp.int32),
            pltpu.VMEM((N_CHUNKS * CHUNK, D), jnp.float32),
            pltpu.VMEM((CHUNK, D), jnp.float32),
            pltpu.SemaphoreType.DMA,
            pltpu.SemaphoreType.DMA,
        ] + [pltpu.SemaphoreType.DMA] * N_CHUNKS,
    )(_emb_body)
    return f(idx, tok_emb, pos_emb)


def kernel(idx, tok_emb, pos_emb):
    return _emb(idx.astype(jnp.int32), tok_emb, pos_emb)


# speedup vs baseline: 1.0453x; 1.0453x over previous
import functools

import jax
import jax.numpy as jnp
from jax import lax
from jax.experimental import pallas as pl
from jax.experimental.pallas import tpu as pltpu
from jax.experimental.pallas import tpu_sc as plsc

BATCH = 4
SEQ = 2048
D = 128
NC, NS = 2, 16
NW = NC * NS
CHUNK = SEQ // NW
N_CHUNKS = BATCH


def _emb_body(idx_hbm, tok_hbm, pos_hbm, out_hbm, idx_v, rows_v, pos_v,
              isem, psem, wsem, *gsems):
    c = lax.axis_index("c")
    s = lax.axis_index("s")
    wid = s * NC + c
    t0 = wid * CHUNK

    icps = [
        pltpu.async_copy(idx_hbm.at[k, pl.ds(t0, CHUNK)], idx_v.at[k], isem)
        for k in range(N_CHUNKS)
    ]

    pos_cp = pltpu.async_copy(pos_hbm.at[0, pl.ds(t0, CHUNK)], pos_v, psem)

    for cp in icps:
        cp.wait()

    gcps = [
        pltpu.async_copy(tok_hbm.at[idx_v.at[k]],
                         rows_v.at[pl.ds(k * CHUNK, CHUNK)], gsems[k])
        for k in range(N_CHUNKS)
    ]
    pos_cp.wait()

    wcps = []
    for k in range(N_CHUNKS):
        gcps[k].wait()

        def add_row(i, carry):
            for j in range(D // 16):
                sl = pl.ds(j * 16, 16)
                plsc.addupdate(rows_v.at[k * CHUNK + i, sl], pos_v[i, sl])
            return carry

        lax.fori_loop(0, CHUNK, add_row, 0)
        wcps.append(
            pltpu.async_copy(rows_v.at[pl.ds(k * CHUNK, CHUNK)],
                             out_hbm.at[k, pl.ds(t0, CHUNK)], wsem))
    for cp in wcps:
        cp.wait()


@jax.jit
def _emb(idx, tok_emb, pos_emb):
    mesh = plsc.VectorSubcoreMesh(core_axis_name="c", subcore_axis_name="s")
    f = functools.partial(
        pl.kernel,
        mesh=mesh,
        out_type=jax.ShapeDtypeStruct((BATCH, SEQ, D), jnp.float32),
        scratch_types=[
            pltpu.VMEM((N_CHUNKS, CHUNK), jnp.int32),
            pltpu.VMEM((N_CHUNKS * CHUNK, D), jnp.float32),
            pltpu.VMEM((CHUNK, D), jnp.float32),
            pltpu.SemaphoreType.DMA,
            pltpu.SemaphoreType.DMA,
            pltpu.SemaphoreType.DMA,
        ] + [pltpu.SemaphoreType.DMA] * N_CHUNKS,
    )(_emb_body)
    return f(idx, tok_emb, pos_emb)


def kernel(idx, tok_emb, pos_emb):
    return _emb(idx.astype(jnp.int32), tok_emb, pos_emb)


# speedup vs baseline: 1.0462x; 1.0009x over previous
import functools

import jax
import jax.numpy as jnp
from jax import lax
from jax.experimental import pallas as pl
from jax.experimental.pallas import tpu as pltpu
from jax.experimental.pallas import tpu_sc as plsc

BATCH = 4
SEQ = 2048
D = 128
NC, NS = 2, 16
NW = NC * NS
CHUNK = SEQ // NW
N_CHUNKS = BATCH


def _emb_body(idx_hbm, tok_hbm, pos_hbm, out_hbm, idx_v, rows_v, pos_v,
              psem, wsem, *gsems):
    c = lax.axis_index("c")
    s = lax.axis_index("s")
    wid = s * NC + c
    t0 = wid * CHUNK

    icps = [
        pltpu.async_copy(idx_hbm.at[k, pl.ds(t0, CHUNK)], idx_v.at[k],
                         gsems[k])
        for k in range(N_CHUNKS)
    ]

    pos_cp = pltpu.async_copy(pos_hbm.at[0, pl.ds(t0, CHUNK)], pos_v, psem)

    gcps = []
    for k in range(N_CHUNKS):
        icps[k].wait()
        gcps.append(
            pltpu.async_copy(tok_hbm.at[idx_v.at[k]],
                             rows_v.at[pl.ds(k * CHUNK, CHUNK)], gsems[k]))
    pos_cp.wait()

    wcps = []
    for k in range(N_CHUNKS):
        gcps[k].wait()

        def add_row(i, carry):
            for j in range(D // 16):
                sl = pl.ds(j * 16, 16)
                plsc.addupdate(rows_v.at[k * CHUNK + i, sl], pos_v[i, sl])
            return carry

        lax.fori_loop(0, CHUNK, add_row, 0, unroll=2)
        wcps.append(
            pltpu.async_copy(rows_v.at[pl.ds(k * CHUNK, CHUNK)],
                             out_hbm.at[k, pl.ds(t0, CHUNK)], wsem))
    for cp in wcps:
        cp.wait()


@jax.jit
def _emb(idx, tok_emb, pos_emb):
    mesh = plsc.VectorSubcoreMesh(core_axis_name="c", subcore_axis_name="s")
    f = functools.partial(
        pl.kernel,
        mesh=mesh,
        out_type=jax.ShapeDtypeStruct((BATCH, SEQ, D), jnp.float32),
        scratch_types=[
            pltpu.VMEM((N_CHUNKS, CHUNK), jnp.int32),
            pltpu.VMEM((N_CHUNKS * CHUNK, D), jnp.float32),
            pltpu.VMEM((CHUNK, D), jnp.float32),
            pltpu.SemaphoreType.DMA,
            pltpu.SemaphoreType.DMA,
        ] + [pltpu.SemaphoreType.DMA] * N_CHUNKS,
    )(_emb_body)
    return f(idx, tok_emb, pos_emb)


def kernel(idx, tok_emb, pos_emb):
    return _emb(idx.astype(jnp.int32), tok_emb, pos_emb)


# speedup vs baseline: 1.0551x; 1.0085x over previous
import functools

import jax
import jax.numpy as jnp
from jax import lax
from jax.experimental import pallas as pl
from jax.experimental.pallas import tpu as pltpu
from jax.experimental.pallas import tpu_sc as plsc

BATCH = 4
SEQ = 2048
D = 128
NC, NS = 2, 16
NW = NC * NS
CHUNK = SEQ // NW
N_CHUNKS = BATCH


def _emb_body(idx_hbm, tok_hbm, pos_hbm, out_hbm, idx_v, rows_v, pos_v,
              psem, wsem, *gsems):
    c = lax.axis_index("c")
    s = lax.axis_index("s")
    wid = s * NC + c
    t0 = wid * CHUNK

    icps = [
        pltpu.async_copy(idx_hbm.at[k, pl.ds(t0, CHUNK)], idx_v.at[k],
                         gsems[k])
        for k in range(N_CHUNKS)
    ]

    pos_cp = pltpu.async_copy(pos_hbm.at[0, pl.ds(t0, CHUNK)], pos_v, psem)

    gcps = []
    for k in range(N_CHUNKS):
        icps[k].wait()
        gcps.append(
            pltpu.async_copy(tok_hbm.at[idx_v.at[k]],
                             rows_v.at[pl.ds(k * CHUNK, CHUNK)], gsems[k]))
    pos_cp.wait()

    wcps = []
    for k in range(N_CHUNKS):
        gcps[k].wait()

        def add_row(i, carry):
            for j in range(D // 16):
                sl = pl.ds(j * 16, 16)
                plsc.addupdate(rows_v.at[k * CHUNK + i, sl], pos_v[i, sl])
            return carry

        lax.fori_loop(0, CHUNK, add_row, 0)
        wcps.append(
            pltpu.async_copy(rows_v.at[pl.ds(k * CHUNK, CHUNK)],
                             out_hbm.at[k, pl.ds(t0, CHUNK)], wsem))
    for cp in wcps:
        cp.wait()


@jax.jit
def _emb(idx, tok_emb, pos_emb):
    mesh = plsc.VectorSubcoreMesh(core_axis_name="c", subcore_axis_name="s")
    f = functools.partial(
        pl.kernel,
        mesh=mesh,
        out_type=jax.ShapeDtypeStruct((BATCH, SEQ, D), jnp.float32),
        scratch_types=[
            pltpu.VMEM((N_CHUNKS, CHUNK), jnp.int32),
            pltpu.VMEM((N_CHUNKS * CHUNK, D), jnp.float32),
            pltpu.VMEM((CHUNK, D), jnp.float32),
            pltpu.SemaphoreType.DMA,
            pltpu.SemaphoreType.DMA,
        ] + [pltpu.SemaphoreType.DMA] * N_CHUNKS,
    )(_emb_body)
    return f(idx, tok_emb, pos_emb)


def kernel(idx, tok_emb, pos_emb):
    return _emb(idx.astype(jnp.int32), tok_emb, pos_emb)


# speedup vs baseline: 1.0712x; 1.0153x over previous
import functools

import jax
import jax.numpy as jnp
from jax import lax
from jax.experimental import pallas as pl
from jax.experimental.pallas import tpu as pltpu
from jax.experimental.pallas import tpu_sc as plsc

BATCH = 4
SEQ = 2048
D = 128
NC, NS = 2, 16
NW = NC * NS
CHUNK = SEQ // NW
N_CHUNKS = BATCH


def _emb_body(idx_hbm, tok_hbm, pos_hbm, out_hbm, idx_v, rows_v, pos_v,
              psem, wsem, *gsems):
    c = lax.axis_index("c")
    s = lax.axis_index("s")
    wid = s * NC + c
    t0 = wid * CHUNK

    icps = [
        pltpu.async_copy(idx_hbm.at[k, pl.ds(t0, CHUNK)], idx_v.at[k],
                         gsems[k])
        for k in range(N_CHUNKS)
    ]

    pos_cp = pltpu.async_copy(pos_hbm.at[0, pl.ds(t0, CHUNK)], pos_v, psem)

    SUB = CHUNK // 2
    N_SUB = 2 * N_CHUNKS

    def sub_src(h):
        return tok_hbm.at[idx_v.at[h // 2, pl.ds((h % 2) * SUB, SUB)]]

    gcps = []
    for h in range(N_CHUNKS):
        icps[h].wait()
        gcps.append(pltpu.async_copy(
            sub_src(h), rows_v.at[pl.ds(h * SUB, SUB)], gsems[h]))
    pos_cp.wait()

    wcps = []
    for h in range(N_SUB):
        gcps[h].wait()
        if h + N_CHUNKS < N_SUB:
            g = h + N_CHUNKS
            gcps.append(pltpu.async_copy(
                sub_src(g), rows_v.at[pl.ds(g * SUB, SUB)],
                gsems[g % N_CHUNKS]))

        def add_row(i, carry):
            for j in range(D // 16):
                sl = pl.ds(j * 16, 16)
                plsc.addupdate(rows_v.at[h * SUB + i, sl],
                               pos_v[(h % 2) * SUB + i, sl])
            return carry

        lax.fori_loop(0, SUB, add_row, 0)
        wcps.append(
            pltpu.async_copy(rows_v.at[pl.ds(h * SUB, SUB)],
                             out_hbm.at[h // 2,
                                        pl.ds(t0 + (h % 2) * SUB, SUB)],
                             wsem))
    for cp in wcps:
        cp.wait()


@jax.jit
def _emb(idx, tok_emb, pos_emb):
    mesh = plsc.VectorSubcoreMesh(core_axis_name="c", subcore_axis_name="s")
    f = functools.partial(
        pl.kernel,
        mesh=mesh,
        out_type=jax.ShapeDtypeStruct((BATCH, SEQ, D), jnp.float32),
        scratch_types=[
            pltpu.VMEM((N_CHUNKS, CHUNK), jnp.int32),
            pltpu.VMEM((N_CHUNKS * CHUNK, D), jnp.float32),
            pltpu.VMEM((CHUNK, D), jnp.float32),
            pltpu.SemaphoreType.DMA,
            pltpu.SemaphoreType.DMA,
        ] + [pltpu.SemaphoreType.DMA] * N_CHUNKS,
    )(_emb_body)
    return f(idx, tok_emb, pos_emb)


def kernel(idx, tok_emb, pos_emb):
    return _emb(idx.astype(jnp.int32), tok_emb, pos_emb)
